# Initial kernel scaffold; baseline (speedup 1.0000x reference)
import jax
import jax.numpy as jnp
from jax.experimental import pallas as pl


def kernel(x, weight, bias, gamma, beta):
    raise NotImplementedError("write your pallas kernel here")


# speedup vs baseline: 1.4054x; 1.4054x over previous
import functools

import jax
import jax.numpy as jnp
from jax import lax
from jax.experimental import pallas as pl
from jax.experimental.pallas import tpu as pltpu


def _patches(x_ref, *, K, pad):
    TN, C_in, L = x_ref.shape
    lane = lax.broadcasted_iota(jnp.int32, (C_in, L), 1)
    keeps = {}
    for k in range(K):
        d = k - pad
        if d != 0:
            keeps[k] = (lane < L - d) if d > 0 else (lane >= -d)
    tiles = []
    for n in range(TN):
        xb = x_ref[n].astype(jnp.bfloat16)
        rows = []
        for k in range(K):
            d = k - pad
            if d == 0:
                rows.append(xb)
            else:
                sh = pltpu.roll(xb, (-d) % L, 1)
                rows.append(jnp.where(keeps[k], sh, jnp.bfloat16(0)))
        tiles.append(jnp.concatenate(rows, axis=0))
    return jnp.concatenate(tiles, axis=1) if TN > 1 else tiles[0]


def _stats_body(x_ref, w_ref, acc_ref, *, K, pad):
    @pl.when(pl.program_id(1) == 0)
    def _init():
        acc_ref[...] = jnp.zeros_like(acc_ref)

    pm = _patches(x_ref, K=K, pad=pad)
    conv = jnp.dot(w_ref[...], pm, preferred_element_type=jnp.float32)
    s = jnp.sum(conv, axis=1, keepdims=True)
    q = jnp.sum(conv * conv, axis=1, keepdims=True)
    acc_ref[0] += jnp.concatenate([s, q], axis=1)


def _apply_body(x_ref, w_ref, shift_ref, o_ref, *, K, pad):
    TN, _, L = x_ref.shape
    pm = _patches(x_ref, K=K, pad=pad)
    conv = jnp.dot(w_ref[...], pm, preferred_element_type=jnp.float32)
    act = jnp.maximum(conv + shift_ref[...], 0.0).astype(o_ref.dtype)
    for n in range(TN):
        o_ref[n] = act[:, n * L:(n + 1) * L]


def kernel(x, weight, bias, gamma, beta):
    del bias
    eps = 1e-5
    pad = 1
    N, C_in, L = x.shape
    C_out, _, K = weight.shape
    assert L + 2 * pad - K + 1 == L, "K=3, pad=1 keeps length"

    wf = jnp.transpose(weight, (0, 2, 1)).reshape(C_out, K * C_in)
    wf16 = wf.astype(jnp.bfloat16)

    TN = 8
    while N % TN:
        TN -= 1
    n_tiles = N // TN
    nsplit = 2 if (n_tiles % 2 == 0 and n_tiles >= 2) else 1
    tps = n_tiles // nsplit
    vmem = 38 * 1024 * 1024

    stats = pl.pallas_call(
        functools.partial(_stats_body, K=K, pad=pad),
        out_shape=jax.ShapeDtypeStruct((nsplit, C_out, 2), jnp.float32),
        grid=(nsplit, tps),
        in_specs=[
            pl.BlockSpec((TN, C_in, L), lambda s, t: (s * tps + t, 0, 0)),
            pl.BlockSpec((C_out, K * C_in), lambda s, t: (0, 0)),
        ],
        out_specs=pl.BlockSpec((1, C_out, 2), lambda s, t: (s, 0, 0)),
        compiler_params=pltpu.CompilerParams(
            dimension_semantics=("parallel", "arbitrary"),
            vmem_limit_bytes=vmem),
    )(x, wf16)

    tot = jnp.sum(stats, axis=0)
    cnt = jnp.float32(N * L)
    mean = tot[:, 0] / cnt
    var = jnp.maximum(tot[:, 1] / cnt - mean * mean, 0.0)
    scale = gamma.astype(jnp.float32) * lax.rsqrt(var + eps)
    shift = (beta.astype(jnp.float32) - mean * scale).reshape(C_out, 1)
    w_bn = (wf.astype(jnp.float32) * scale[:, None]).astype(jnp.bfloat16)

    out = pl.pallas_call(
        functools.partial(_apply_body, K=K, pad=pad),
        out_shape=jax.ShapeDtypeStruct((N, C_out, L), x.dtype),
        grid=(n_tiles,),
        in_specs=[
            pl.BlockSpec((TN, C_in, L), lambda t: (t, 0, 0)),
            pl.BlockSpec((C_out, K * C_in), lambda t: (0, 0)),
            pl.BlockSpec((C_out, 1), lambda t: (0, 0)),
        ],
        out_specs=pl.BlockSpec((TN, C_out, L), lambda t: (t, 0, 0)),
        compiler_params=pltpu.CompilerParams(
            dimension_semantics=("parallel",),
            vmem_limit_bytes=vmem),
    )(x, w_bn, shift)
    return out


# speedup vs baseline: 1.5900x; 1.1313x over previous
import functools

import jax
import jax.numpy as jnp
from jax import lax
from jax.experimental import pallas as pl
from jax.experimental.pallas import tpu as pltpu


def _patches(x_ref, *, K, pad):
    TN, C_in, L = x_ref.shape
    lane = lax.broadcasted_iota(jnp.int32, (C_in, L), 1)
    keeps = {}
    for k in range(K):
        d = k - pad
        if d != 0:
            keeps[k] = (lane < L - d) if d > 0 else (lane >= -d)
    tiles = []
    for n in range(TN):
        xb = x_ref[n].astype(jnp.bfloat16)
        rows = []
        for k in range(K):
            d = k - pad
            if d == 0:
                rows.append(xb)
            else:
                sh = pltpu.roll(xb, (-d) % L, 1)
                rows.append(jnp.where(keeps[k], sh, jnp.bfloat16(0)))
        tiles.append(jnp.concatenate(rows, axis=0))
    return jnp.concatenate(tiles, axis=1) if TN > 1 else tiles[0]


def _stats_body(x_ref, w_ref, acc_ref, *, K, pad):
    @pl.when(pl.program_id(1) == 0)
    def _init():
        acc_ref[...] = jnp.zeros_like(acc_ref)

    pm = _patches(x_ref, K=K, pad=pad)
    conv = jnp.dot(w_ref[...], pm, preferred_element_type=jnp.float32)
    s = jnp.sum(conv, axis=1, keepdims=True)
    q = jnp.sum(conv * conv, axis=1, keepdims=True)
    acc_ref[0] += jnp.concatenate([s, q], axis=1)


def _apply_body(x_ref, w_ref, stats_ref, gb_ref, o_ref, *, K, pad, count, eps):
    TN, _, L = x_ref.shape
    tot = jnp.sum(stats_ref[...], axis=0)
    mean = tot[:, 0:1] / count
    var = jnp.maximum(tot[:, 1:2] / count - mean * mean, 0.0)
    gamma = gb_ref[:, 0:1]
    beta = gb_ref[:, 1:2]
    scale = gamma * lax.rsqrt(var + eps)
    shift = beta - mean * scale
    w = (w_ref[...].astype(jnp.float32) * scale).astype(jnp.bfloat16)

    pm = _patches(x_ref, K=K, pad=pad)
    conv = jnp.dot(w, pm, preferred_element_type=jnp.float32)
    act = jnp.maximum(conv + shift, 0.0).astype(o_ref.dtype)
    for n in range(TN):
        o_ref[n] = act[:, n * L:(n + 1) * L]


def kernel(x, weight, bias, gamma, beta):
    del bias
    eps = 1e-5
    pad = 1
    N, C_in, L = x.shape
    C_out, _, K = weight.shape
    assert L + 2 * pad - K + 1 == L, "K=3, pad=1 keeps length"

    wf16 = jnp.transpose(weight, (0, 2, 1)).reshape(C_out, K * C_in)
    wf16 = wf16.astype(jnp.bfloat16)
    gb = jnp.stack([gamma, beta], axis=1).astype(jnp.float32)

    TN = 16
    while N % TN:
        TN -= 1
    n_tiles = N // TN
    nsplit = 2 if (n_tiles % 2 == 0 and n_tiles >= 2) else 1
    tps = n_tiles // nsplit
    vmem = 52 * 1024 * 1024

    stats = pl.pallas_call(
        functools.partial(_stats_body, K=K, pad=pad),
        out_shape=jax.ShapeDtypeStruct((nsplit, C_out, 2), jnp.float32),
        grid=(nsplit, tps),
        in_specs=[
            pl.BlockSpec((TN, C_in, L), lambda s, t: (s * tps + t, 0, 0)),
            pl.BlockSpec((C_out, K * C_in), lambda s, t: (0, 0)),
        ],
        out_specs=pl.BlockSpec((1, C_out, 2), lambda s, t: (s, 0, 0)),
        compiler_params=pltpu.CompilerParams(
            dimension_semantics=("parallel", "arbitrary"),
            vmem_limit_bytes=vmem),
    )(x, wf16)

    out = pl.pallas_call(
        functools.partial(_apply_body, K=K, pad=pad,
                          count=float(N * L), eps=eps),
        out_shape=jax.ShapeDtypeStruct((N, C_out, L), x.dtype),
        grid=(n_tiles,),
        in_specs=[
            pl.BlockSpec((TN, C_in, L), lambda t: (t, 0, 0)),
            pl.BlockSpec((C_out, K * C_in), lambda t: (0, 0)),
            pl.BlockSpec((nsplit, C_out, 2), lambda t: (0, 0, 0)),
            pl.BlockSpec((C_out, 2), lambda t: (0, 0)),
        ],
        out_specs=pl.BlockSpec((TN, C_out, L), lambda t: (t, 0, 0)),
        compiler_params=pltpu.CompilerParams(
            dimension_semantics=("parallel",),
            vmem_limit_bytes=vmem),
    )(x, wf16, stats, gb)
    return out
